# Initial kernel scaffold; baseline (speedup 1.0000x reference)
import jax
import jax.numpy as jnp
from jax.experimental import pallas as pl


def kernel(x, weight):
    raise NotImplementedError("write your pallas kernel here")


# speedup vs baseline: 1.8732x; 1.8732x over previous
import jax
import jax.numpy as jnp
from jax import lax
from jax.experimental import pallas as pl
from jax.experimental.pallas import tpu as pltpu
from jax.experimental.pallas import tpu_sc as plsc

_NUM_CORES = 2
_NUM_SUBCORES = 16
_NW = _NUM_CORES * _NUM_SUBCORES

_B, _H, _D = 16384, 50, 64
_TOTAL = _B * _H
_PER_W = _TOTAL // _NW
_CHUNK = 128
_NCHUNK = _PER_W // _CHUNK
_NBUF = 8


def _emb_body(idx_hbm, table_hbm, out_hbm, idx_v, rows_v, gsem, ssem):
    wid = lax.axis_index("s") * _NUM_CORES + lax.axis_index("c")
    base = wid * _PER_W
    pltpu.sync_copy(idx_hbm.at[wid], idx_v)

    def _gather(c, b):
        return pltpu.make_async_copy(
            table_hbm.at[idx_v.at[c]], rows_v.at[b], gsem.at[b])

    def _store(c, b):
        return pltpu.make_async_copy(
            rows_v.at[b], out_hbm.at[pl.ds(base + c * _CHUNK, _CHUNK)],
            ssem.at[b])

    for b in range(_NBUF):
        _gather(b, b).start()

    @pl.loop(_NBUF, _NCHUNK, step=_NBUF)
    def _(i):
        for b in range(_NBUF):
            c_prev = i - _NBUF + b
            _gather(c_prev, b).wait()
            _store(c_prev, b).start()
        for b in range(_NBUF):
            c_prev = i - _NBUF + b
            _store(c_prev, b).wait()
            _gather(i + b, b).start()

    for b in range(_NBUF):
        c_prev = _NCHUNK - _NBUF + b
        _gather(c_prev, b).wait()
        _store(c_prev, b).start()
    for b in range(_NBUF):
        c_prev = _NCHUNK - _NBUF + b
        _store(c_prev, b).wait()


_mesh = plsc.VectorSubcoreMesh(
    core_axis_name="c", subcore_axis_name="s",
    num_cores=_NUM_CORES, num_subcores=_NUM_SUBCORES)

_emb_call = pl.kernel(
    _emb_body,
    out_type=jax.ShapeDtypeStruct((_TOTAL, _D), jnp.float32),
    mesh=_mesh,
    scratch_types=[
        pltpu.VMEM((_NCHUNK, _CHUNK), jnp.int32),
        pltpu.VMEM((_NBUF, _CHUNK, _D), jnp.float32),
        pltpu.SemaphoreType.DMA((_NBUF,)),
        pltpu.SemaphoreType.DMA((_NBUF,)),
    ],
    compiler_params=pltpu.CompilerParams(use_tc_tiling_on_sc=False),
)


def kernel(x, weight):
    idx = x.reshape(_NW, _NCHUNK, _CHUNK)
    out = _emb_call(idx, weight)
    return out.reshape(_B, _H, _D)


# speedup vs baseline: 1.8764x; 1.0017x over previous
import jax
import jax.numpy as jnp
from jax import lax
from jax.experimental import pallas as pl
from jax.experimental.pallas import tpu as pltpu
from jax.experimental.pallas import tpu_sc as plsc

_NUM_CORES = 2
_NUM_SUBCORES = 16
_NW = _NUM_CORES * _NUM_SUBCORES

_B, _H, _D = 16384, 50, 64
_TOTAL = _B * _H
_PER_W = _TOTAL // _NW
_CHUNK = 128
_NCHUNK = _PER_W // _CHUNK
_NBUF = 10
_LEAD = 5


def _emb_body(idx_hbm, table_hbm, out_hbm, idx_v, rows_v, gsem, ssem):
    wid = lax.axis_index("s") * _NUM_CORES + lax.axis_index("c")
    base = wid * _PER_W
    pltpu.sync_copy(idx_hbm.at[wid], idx_v)

    def _gather(c, b):
        return pltpu.make_async_copy(
            table_hbm.at[idx_v.at[c]], rows_v.at[b], gsem.at[b])

    def _store(c, b):
        return pltpu.make_async_copy(
            rows_v.at[b], out_hbm.at[pl.ds(base + c * _CHUNK, _CHUNK)],
            ssem.at[b])


    for c in range(_NBUF):
        if c >= _LEAD:
            _gather(c - _LEAD, c - _LEAD).wait()
            _store(c - _LEAD, c - _LEAD).start()
        _gather(c, c).start()

    @pl.loop(_NBUF, _NCHUNK, step=_NBUF)
    def _(i):
        for b in range(_NBUF):
            c = i + b
            bs = (b - _LEAD) % _NBUF
            _gather(c - _LEAD, bs).wait()
            _store(c - _LEAD, bs).start()
            _store(c - _NBUF, b).wait()
            _gather(c, b).start()

    for c in range(_NCHUNK, _NCHUNK + _LEAD):
        bs = (c - _LEAD) % _NBUF
        _gather(c - _LEAD, bs).wait()
        _store(c - _LEAD, bs).start()
    for b in range(_NBUF):
        _store(_NCHUNK - _NBUF + b, b).wait()


_mesh = plsc.VectorSubcoreMesh(
    core_axis_name="c", subcore_axis_name="s",
    num_cores=_NUM_CORES, num_subcores=_NUM_SUBCORES)

_emb_call = pl.kernel(
    _emb_body,
    out_type=jax.ShapeDtypeStruct((_TOTAL, _D), jnp.float32),
    mesh=_mesh,
    scratch_types=[
        pltpu.VMEM((_NCHUNK, _CHUNK), jnp.int32),
        pltpu.VMEM((_NBUF, _CHUNK, _D), jnp.float32),
        pltpu.SemaphoreType.DMA((_NBUF,)),
        pltpu.SemaphoreType.DMA((_NBUF,)),
    ],
    compiler_params=pltpu.CompilerParams(use_tc_tiling_on_sc=False),
)


def kernel(x, weight):
    idx = x.reshape(_NW, _NCHUNK, _CHUNK)
    out = _emb_call(idx, weight)
    return out.reshape(_B, _H, _D)
